# Initial kernel scaffold; baseline (speedup 1.0000x reference)
import jax
import jax.numpy as jnp
from jax.experimental import pallas as pl


def kernel(x, table):
    raise NotImplementedError("write your pallas kernel here")


# speedup vs baseline: 13.1239x; 13.1239x over previous
import functools

import jax
import jax.numpy as jnp
from jax import lax
from jax.experimental import pallas as pl
from jax.experimental.pallas import tpu as pltpu
from jax.experimental.pallas import tpu_sc as plsc

B = 16384
L = 200
D = 64
NC = 2
NS = 16
NW = NC * NS
RPW = B // NW
CB = 2
NCHUNK = RPW // CB
SUB = 100
NSUB = CB * L // SUB
ND = D // 16


def _body(x_hbm, table_hbm, out_hbm, idx_v, rows_v, out_v, sem):
    wid = lax.axis_index("s") * NC + lax.axis_index("c")
    row_base = wid * RPW

    def chunk_body(c, carry):
        r0 = row_base + c * CB
        pltpu.sync_copy(x_hbm.at[pl.ds(r0 * (L // SUB), NSUB), :], idx_v)
        cps = [
            pltpu.async_copy(
                table_hbm.at[idx_v.at[j]],
                rows_v.at[pl.ds(j * SUB, SUB)],
                sem,
            )
            for j in range(NSUB)
        ]
        for cp in cps:
            cp.wait()
        for b in range(CB):
            def t_body(t, accs):
                base = b * L + t
                return tuple(
                    accs[d] + rows_v[base, pl.ds(d * 16, 16)]
                    for d in range(ND)
                )
            accs = lax.fori_loop(
                0, L, t_body,
                tuple(jnp.zeros((16,), jnp.float32) for _ in range(ND)),
            )
            for d in range(ND):
                out_v[b, pl.ds(d * 16, 16)] = accs[d] * jnp.float32(1.0 / L)
        pltpu.sync_copy(out_v, out_hbm.at[pl.ds(r0, CB), :])
        return carry

    lax.fori_loop(0, NCHUNK, chunk_body, 0)


@functools.partial(
    pl.kernel,
    mesh=plsc.VectorSubcoreMesh(core_axis_name="c", subcore_axis_name="s"),
    out_type=jax.ShapeDtypeStruct((B, D), jnp.float32),
    scratch_types=[
        pltpu.VMEM((NSUB, SUB), jnp.int32),
        pltpu.VMEM((CB * L, D), jnp.float32),
        pltpu.VMEM((CB, D), jnp.float32),
        pltpu.SemaphoreType.DMA,
    ],
    compiler_params=pltpu.CompilerParams(use_tc_tiling_on_sc=False),
)
def _pooled_lookup(x_hbm, table_hbm, out_hbm, idx_v, rows_v, out_v, sem):
    _body(x_hbm, table_hbm, out_hbm, idx_v, rows_v, out_v, sem)


@jax.jit
def kernel(x, table):
    return _pooled_lookup(x.reshape(B * L // SUB, SUB), table)


# speedup vs baseline: 27.9032x; 2.1261x over previous
import functools

import jax
import jax.numpy as jnp
from jax import lax
from jax.experimental import pallas as pl
from jax.experimental.pallas import tpu as pltpu
from jax.experimental.pallas import tpu_sc as plsc

B = 16384
L = 200
D = 64
NC = 2
NS = 16
NW = NC * NS
RPW = B // NW
CB = 4
NCHUNK = RPW // CB
SUB = 100
NSUB = CB * L // SUB
ND = D // 16
UNROLL = 2


def _body(x_hbm, table_hbm, out_hbm, idx_v, rows_v, out_v, sem0, sem1):
    wid = lax.axis_index("s") * NC + lax.axis_index("c")
    row_base = wid * RPW
    sems = (sem0, sem1)

    def fire(slot, c):
        r0 = row_base + c * CB
        pltpu.sync_copy(
            x_hbm.at[pl.ds(r0 * (L // SUB), NSUB), :], idx_v.at[slot]
        )
        for j in range(NSUB):
            pltpu.async_copy(
                table_hbm.at[idx_v.at[slot].at[j]],
                rows_v.at[slot].at[pl.ds(j * SUB, SUB)],
                sems[slot],
            )

    def drain(slot):
        for j in range(NSUB):
            pltpu.make_async_copy(
                table_hbm.at[idx_v.at[slot].at[j]],
                rows_v.at[slot].at[pl.ds(j * SUB, SUB)],
                sems[slot],
            ).wait()

    def reduce_store(slot, c):
        r0 = row_base + c * CB
        rows = rows_v.at[slot]
        for b in range(CB):
            def t_body(t, accs):
                base = b * L + UNROLL * t
                for u in range(UNROLL):
                    accs = tuple(
                        accs[d] + rows[base + u, pl.ds(d * 16, 16)]
                        for d in range(ND)
                    )
                return accs
            accs = lax.fori_loop(
                0, L // UNROLL, t_body,
                tuple(jnp.zeros((16,), jnp.float32) for _ in range(ND)),
            )
            for d in range(ND):
                out_v[b, pl.ds(d * 16, 16)] = accs[d] * jnp.float32(1.0 / L)
        pltpu.sync_copy(out_v, out_hbm.at[pl.ds(r0, CB), :])

    fire(0, 0)

    def pair_body(k, carry):
        c0 = 2 * k
        fire(1, c0 + 1)
        drain(0)
        reduce_store(0, c0)

        @pl.when(c0 + 2 < NCHUNK)
        def _():
            fire(0, c0 + 2)

        drain(1)
        reduce_store(1, c0 + 1)
        return carry

    lax.fori_loop(0, NCHUNK // 2, pair_body, 0)


@functools.partial(
    pl.kernel,
    mesh=plsc.VectorSubcoreMesh(core_axis_name="c", subcore_axis_name="s"),
    out_type=jax.ShapeDtypeStruct((B, D), jnp.float32),
    scratch_types=[
        pltpu.VMEM((2, NSUB, SUB), jnp.int32),
        pltpu.VMEM((2, CB * L, D), jnp.float32),
        pltpu.VMEM((CB, D), jnp.float32),
        pltpu.SemaphoreType.DMA,
        pltpu.SemaphoreType.DMA,
    ],
    compiler_params=pltpu.CompilerParams(use_tc_tiling_on_sc=False),
)
def _pooled_lookup(x_hbm, table_hbm, out_hbm, idx_v, rows_v, out_v, sem0, sem1):
    _body(x_hbm, table_hbm, out_hbm, idx_v, rows_v, out_v, sem0, sem1)


@jax.jit
def kernel(x, table):
    return _pooled_lookup(x.reshape(B * L // SUB, SUB), table)
